# baseline (speedup 1.0000x reference)
import functools

import jax
import jax.numpy as jnp
import numpy as np
from jax import lax
from jax.experimental import pallas as pl
from jax.experimental.pallas import tpu as pltpu
from jax.experimental.pallas import tpu_sc as plsc

N_NODES = 10000
N_EDGES = 320000
D_FEAT = 128
C = 1.0
MIN_NORM = 1e-15
EPS = 1e-5

NUM_CORES = 2
NUM_SUBCORES = 16
NUM_TILES = NUM_CORES * NUM_SUBCORES
EDGES_PER_TILE = N_EDGES // NUM_TILES
CHUNK = 80
N_CHUNKS = EDGES_PER_TILE // CHUNK
ROWS_PER_TILE = 624
ROWS_TAIL = N_NODES - ROWS_PER_TILE * NUM_SUBCORES
LANES = 16
PAIRS = D_FEAT // 32

_PERM = np.empty(D_FEAT, np.int32)
for _k in range(PAIRS):
    for _i in range(LANES):
        _PERM[32 * _k + 2 * _i] = 32 * _k + _i
        _PERM[32 * _k + 2 * _i + 1] = 32 * _k + LANES + _i


def _sc_aggregate(x_packed, row_idx, col_idx, vals, zeros_blk):
    mesh = plsc.VectorSubcoreMesh(
        core_axis_name="c", subcore_axis_name="s")

    @functools.partial(
        pl.kernel,
        out_type=jax.ShapeDtypeStruct((NUM_CORES, N_NODES, D_FEAT),
                                      jnp.float32),
        mesh=mesh,
        compiler_params=pltpu.CompilerParams(use_tc_tiling_on_sc=False),
        scratch_types=[
            pltpu.VMEM((EDGES_PER_TILE,), jnp.int32),
            pltpu.VMEM((2, CHUNK), jnp.int32),
            pltpu.VMEM((EDGES_PER_TILE,), jnp.float32),
            pltpu.VMEM((CHUNK, D_FEAT // 2), jnp.int32),
            pltpu.VMEM((CHUNK, D_FEAT // 2), jnp.int32),
            pltpu.VMEM((CHUNK, D_FEAT), jnp.float32),
            pltpu.VMEM((CHUNK, D_FEAT), jnp.float32),
            pltpu.VMEM_SHARED((N_NODES, D_FEAT), jnp.float32),
            pltpu.SemaphoreType.DMA,
            pltpu.SemaphoreType.DMA,
            pltpu.SemaphoreType.DMA,
            pltpu.SemaphoreType.DMA,
            pltpu.SemaphoreType.DMA,
            pltpu.SemaphoreType.DMA,
            pltpu.SemaphoreType.DMA,
        ],
    )
    def agg(x_hbm, row_hbm, col_hbm, val_hbm, zero_hbm, out_hbm,
            colv, rowv, valv, gb_a, gb_b, sb_a, sb_b, acc, sem_a, sem_b,
            rsem_a, rsem_b, ssem_a, ssem_b, isem):
        cid = lax.axis_index("c")
        sid = lax.axis_index("s")
        tid = sid * NUM_CORES + cid

        ebase = tid * EDGES_PER_TILE
        d1 = pltpu.async_copy(col_hbm.at[pl.ds(ebase, EDGES_PER_TILE)],
                              colv, isem)
        d3 = pltpu.async_copy(val_hbm.at[pl.ds(ebase, EDGES_PER_TILE)],
                              valv, isem)

        pltpu.sync_copy(zero_hbm, acc.at[pl.ds(sid * ROWS_PER_TILE,
                                               ROWS_PER_TILE)])

        @pl.when(sid == NUM_SUBCORES - 1)
        def _zero_tail():
            pltpu.sync_copy(
                zero_hbm.at[pl.ds(0, ROWS_TAIL)],
                acc.at[pl.ds(NUM_SUBCORES * ROWS_PER_TILE, ROWS_TAIL)])

        d1.wait()
        d3.wait()
        plsc.subcore_barrier()

        def row_start(i, par, rsem):
            pltpu.async_copy(row_hbm.at[pl.ds(ebase + i * CHUNK, CHUNK)],
                             rowv.at[par], rsem)

        def row_wait(i, par, rsem):
            pltpu.make_async_copy(
                row_hbm.at[pl.ds(ebase + i * CHUNK, CHUNK)],
                rowv.at[par], rsem).wait()

        def gather_start(i, gb, sem):
            pltpu.async_copy(x_hbm.at[colv.at[pl.ds(i * CHUNK, CHUNK)]],
                             gb, sem)

        def gather_wait(i, gb, sem):
            pltpu.make_async_copy(x_hbm.at[colv.at[pl.ds(i * CHUNK, CHUNK)]],
                                  gb, sem).wait()

        def scale(i, gb, sb):
            def g(gi, c2):
                vv = valv[pl.ds(i * CHUNK + gi * LANES, LANES)]
                for j in range(LANES):
                    v = vv[j]
                    e = gi * LANES + j
                    for k in range(PAIRS):
                        w = gb[e, pl.ds(k * LANES, LANES)]
                        lo = lax.bitcast_convert_type(w << 16, jnp.float32)
                        hi = lax.bitcast_convert_type(w & jnp.int32(-65536),
                                                      jnp.float32)
                        sb[e, pl.ds(k * 32, LANES)] = lo * v
                        sb[e, pl.ds(k * 32 + LANES, LANES)] = hi * v
                return c2

            lax.fori_loop(0, CHUNK // LANES, g, 0, unroll=False)

        def scatter_start(par, sb, ssem):
            pltpu.async_copy(sb, acc.at[rowv.at[par]], ssem, add=True)

        def scatter_wait(par, sb, ssem):
            pltpu.make_async_copy(sb, acc.at[rowv.at[par]], ssem).wait()

        gather_start(0, gb_a, sem_a)
        gather_start(1, gb_b, sem_b)
        row_start(0, 0, rsem_a)
        row_start(1, 1, rsem_b)

        def pair_body(p, carry):
            i0 = 2 * p
            i1 = 2 * p + 1

            gather_wait(i0, gb_a, sem_a)
            scale(i0, gb_a, sb_a)
            gather_start(i0 + 2, gb_a, sem_a)
            row_wait(i0, 0, rsem_a)
            scatter_start(0, sb_a, ssem_a)

            @pl.when(p > 0)
            def _drain_b():
                scatter_wait(1, sb_b, ssem_b)
                row_start(i1, 1, rsem_b)

            gather_wait(i1, gb_b, sem_b)
            scale(i1, gb_b, sb_b)

            @pl.when(i1 + 2 < N_CHUNKS)
            def _next_b():
                gather_start(i1 + 2, gb_b, sem_b)

            scatter_wait(0, sb_a, ssem_a)
            row_start(i0 + 2, 0, rsem_a)
            row_wait(i1, 1, rsem_b)
            scatter_start(1, sb_b, ssem_b)
            return carry

        lax.fori_loop(0, (N_CHUNKS - 1) // 2, pair_body, 0, unroll=False)
        last = N_CHUNKS - 1
        gather_wait(last, gb_a, sem_a)
        scale(last, gb_a, sb_a)
        scatter_wait(1, sb_b, ssem_b)
        row_wait(last, 0, rsem_a)
        scatter_start(0, sb_a, ssem_a)
        scatter_wait(0, sb_a, ssem_a)
        plsc.subcore_barrier()

        sl = pl.ds(sid * ROWS_PER_TILE, ROWS_PER_TILE)
        pltpu.sync_copy(acc.at[sl], out_hbm.at[cid, sl])

        @pl.when(sid == NUM_SUBCORES - 1)
        def _dump_tail():
            tl = pl.ds(NUM_SUBCORES * ROWS_PER_TILE, ROWS_TAIL)
            pltpu.sync_copy(acc.at[tl], out_hbm.at[cid, tl])

    return agg(x_packed, row_idx, col_idx, vals, zeros_blk)


def _hyper_body(p_ref, o_ref):
    s = p_ref[0] + p_ref[1]
    sqrt_c = jnp.sqrt(C)
    nsq = jnp.sum(s * s, axis=-1, keepdims=True)
    u_norm = jnp.maximum(jnp.sqrt(nsq), MIN_NORM)
    gamma = jnp.tanh(sqrt_c * u_norm) * s / (sqrt_c * u_norm)
    gsq = jnp.sum(gamma * gamma, axis=-1, keepdims=True)
    g_norm = jnp.maximum(jnp.sqrt(gsq), MIN_NORM)
    maxnorm = (1.0 - EPS) / sqrt_c
    o_ref[...] = jnp.where(g_norm > maxnorm, gamma / g_norm * maxnorm, gamma)


def _hyper_project(partials):
    blk = 1000
    grid = N_NODES // blk
    return pl.pallas_call(
        _hyper_body,
        grid=(grid,),
        in_specs=[pl.BlockSpec((NUM_CORES, blk, D_FEAT),
                               lambda i: (0, i, 0))],
        out_specs=pl.BlockSpec((blk, D_FEAT), lambda i: (i, 0)),
        out_shape=jax.ShapeDtypeStruct((N_NODES, D_FEAT), jnp.float32),
    )(partials)


def kernel(x_tangent, adj_indices, adj_values):
    idx = adj_indices.astype(jnp.int32)
    row_idx = idx[0]
    col_idx = idx[1]
    xb = x_tangent.astype(jnp.bfloat16)[:, _PERM]
    x_packed = jax.lax.bitcast_convert_type(
        xb.reshape(N_NODES, D_FEAT // 2, 2), jnp.int32)
    zeros_blk = jnp.zeros((ROWS_PER_TILE, D_FEAT), jnp.float32)
    partials = _sc_aggregate(x_packed, row_idx, col_idx, adj_values,
                             zeros_blk)
    return _hyper_project(partials)


# speedup vs baseline: 1.1079x; 1.1079x over previous
import functools

import jax
import jax.numpy as jnp
from jax import lax
from jax.experimental import pallas as pl
from jax.experimental.pallas import tpu as pltpu
from jax.experimental.pallas import tpu_sc as plsc

N_NODES = 10000
N_EDGES = 320000
D_FEAT = 128
C = 1.0
MIN_NORM = 1e-15
EPS = 1e-5

NUM_CORES = 2
NUM_SUBCORES = 16
NUM_TILES = NUM_CORES * NUM_SUBCORES
CHUNK = 112
N_CHUNKS = 90
EDGES_PER_TILE = CHUNK * N_CHUNKS
N_EDGES_PAD = EDGES_PER_TILE * NUM_TILES
ROWS_PER_TILE = 624
ROWS_TAIL = N_NODES - ROWS_PER_TILE * NUM_SUBCORES
LANES = 16
VPR = D_FEAT // LANES


def _sc_aggregate(x_tangent, row_idx, col_idx, vals, zeros_blk):
    mesh = plsc.VectorSubcoreMesh(
        core_axis_name="c", subcore_axis_name="s")

    @functools.partial(
        pl.kernel,
        out_type=jax.ShapeDtypeStruct((NUM_CORES, N_NODES, D_FEAT),
                                      jnp.float32),
        mesh=mesh,
        scratch_types=[
            pltpu.VMEM((EDGES_PER_TILE,), jnp.int32),
            pltpu.VMEM((2, CHUNK), jnp.int32),
            pltpu.VMEM((EDGES_PER_TILE,), jnp.float32),
            pltpu.VMEM((CHUNK, D_FEAT), jnp.float32),
            pltpu.VMEM((CHUNK, D_FEAT), jnp.float32),
            pltpu.VMEM_SHARED((N_NODES, D_FEAT), jnp.float32),
            pltpu.SemaphoreType.DMA,
            pltpu.SemaphoreType.DMA,
            pltpu.SemaphoreType.DMA,
            pltpu.SemaphoreType.DMA,
            pltpu.SemaphoreType.DMA,
            pltpu.SemaphoreType.DMA,
            pltpu.SemaphoreType.DMA,
        ],
    )
    def agg(x_hbm, row_hbm, col_hbm, val_hbm, zero_hbm, out_hbm,
            colv, rowv, valv, buf_a, buf_b, acc, sem_a, sem_b,
            rsem_a, rsem_b, ssem_a, ssem_b, isem):
        cid = lax.axis_index("c")
        sid = lax.axis_index("s")
        tid = sid * NUM_CORES + cid

        ebase = tid * EDGES_PER_TILE
        d1 = pltpu.async_copy(col_hbm.at[pl.ds(ebase, EDGES_PER_TILE)],
                              colv, isem)
        d3 = pltpu.async_copy(val_hbm.at[pl.ds(ebase, EDGES_PER_TILE)],
                              valv, isem)

        for _h in range(2):
            pltpu.sync_copy(
                zero_hbm.at[pl.ds(_h * (ROWS_PER_TILE // 2),
                                  ROWS_PER_TILE // 2)],
                acc.at[pl.ds(sid * ROWS_PER_TILE + _h * (ROWS_PER_TILE // 2),
                             ROWS_PER_TILE // 2)])

        @pl.when(sid == NUM_SUBCORES - 1)
        def _zero_tail():
            pltpu.sync_copy(
                zero_hbm.at[pl.ds(0, ROWS_TAIL)],
                acc.at[pl.ds(NUM_SUBCORES * ROWS_PER_TILE, ROWS_TAIL)])

        d1.wait()
        d3.wait()
        plsc.subcore_barrier()

        def row_start(i, par, rsem):
            pltpu.async_copy(row_hbm.at[pl.ds(ebase + i * CHUNK, CHUNK)],
                             rowv.at[par], rsem)

        def row_wait(i, par, rsem):
            pltpu.make_async_copy(
                row_hbm.at[pl.ds(ebase + i * CHUNK, CHUNK)],
                rowv.at[par], rsem).wait()

        def gather_start(i, buf, sem):
            pltpu.async_copy(x_hbm.at[colv.at[pl.ds(i * CHUNK, CHUNK)]],
                             buf, sem)

        def gather_wait(i, buf, sem):
            pltpu.make_async_copy(x_hbm.at[colv.at[pl.ds(i * CHUNK, CHUNK)]],
                                  buf, sem).wait()

        def scale(i, buf):
            def g(gi, c2):
                vv = valv[pl.ds(i * CHUNK + gi * LANES, LANES)]
                for j in range(LANES):
                    v = vv[j]
                    e = gi * LANES + j
                    for k in range(VPR):
                        sl = pl.ds(k * LANES, LANES)
                        buf[e, sl] = buf[e, sl] * v
                return c2

            lax.fori_loop(0, CHUNK // LANES, g, 0, unroll=False)

        def scatter_start(par, buf, ssem):
            pltpu.async_copy(buf, acc.at[rowv.at[par]], ssem, add=True)

        def scatter_wait(par, buf, ssem):
            pltpu.make_async_copy(buf, acc.at[rowv.at[par]], ssem).wait()

        gather_start(0, buf_a, sem_a)
        row_start(0, 0, rsem_a)

        def pair_body(p, carry):
            i0 = 2 * p
            i1 = 2 * p + 1

            @pl.when(p > 0)
            def _drain_b():
                scatter_wait(1, buf_b, ssem_b)

            gather_start(i1, buf_b, sem_b)
            row_start(i1, 1, rsem_b)
            gather_wait(i0, buf_a, sem_a)
            scale(i0, buf_a)
            row_wait(i0, 0, rsem_a)
            scatter_start(0, buf_a, ssem_a)
            gather_wait(i1, buf_b, sem_b)
            scale(i1, buf_b)
            scatter_wait(0, buf_a, ssem_a)

            @pl.when(i1 + 1 < N_CHUNKS)
            def _next_a():
                gather_start(i1 + 1, buf_a, sem_a)
                row_start(i1 + 1, 0, rsem_a)

            row_wait(i1, 1, rsem_b)
            scatter_start(1, buf_b, ssem_b)
            return carry

        lax.fori_loop(0, N_CHUNKS // 2, pair_body, 0, unroll=False)
        scatter_wait(1, buf_b, ssem_b)
        plsc.subcore_barrier()

        for _h in range(2):
            _sl = pl.ds(sid * ROWS_PER_TILE + _h * (ROWS_PER_TILE // 2),
                        ROWS_PER_TILE // 2)
            pltpu.sync_copy(acc.at[_sl], out_hbm.at[cid, _sl])

        @pl.when(sid == NUM_SUBCORES - 1)
        def _dump_tail():
            tl = pl.ds(NUM_SUBCORES * ROWS_PER_TILE, ROWS_TAIL)
            pltpu.sync_copy(acc.at[tl], out_hbm.at[cid, tl])

    return agg(x_tangent, row_idx, col_idx, vals, zeros_blk)


def _hyper_body(p_ref, o_ref):
    s = p_ref[0] + p_ref[1]
    sqrt_c = jnp.sqrt(C)
    nsq = jnp.sum(s * s, axis=-1, keepdims=True)
    u_norm = jnp.maximum(jnp.sqrt(nsq), MIN_NORM)
    gamma = jnp.tanh(sqrt_c * u_norm) * s / (sqrt_c * u_norm)
    gsq = jnp.sum(gamma * gamma, axis=-1, keepdims=True)
    g_norm = jnp.maximum(jnp.sqrt(gsq), MIN_NORM)
    maxnorm = (1.0 - EPS) / sqrt_c
    o_ref[...] = jnp.where(g_norm > maxnorm, gamma / g_norm * maxnorm, gamma)


def _hyper_project(partials):
    blk = 1000
    grid = N_NODES // blk
    return pl.pallas_call(
        _hyper_body,
        grid=(grid,),
        in_specs=[pl.BlockSpec((NUM_CORES, blk, D_FEAT),
                               lambda i: (0, i, 0))],
        out_specs=pl.BlockSpec((blk, D_FEAT), lambda i: (i, 0)),
        out_shape=jax.ShapeDtypeStruct((N_NODES, D_FEAT), jnp.float32),
    )(partials)


def kernel(x_tangent, adj_indices, adj_values):
    idx = adj_indices.astype(jnp.int32)
    pad = N_EDGES_PAD - N_EDGES
    row_idx = jnp.concatenate([idx[0], jnp.zeros((pad,), jnp.int32)])
    col_idx = jnp.concatenate([idx[1], jnp.zeros((pad,), jnp.int32)])
    vals = jnp.concatenate([adj_values, jnp.zeros((pad,), jnp.float32)])
    zeros_blk = jnp.zeros((ROWS_PER_TILE, D_FEAT), jnp.float32)
    partials = _sc_aggregate(x_tangent, row_idx, col_idx, vals,
                             zeros_blk)
    return _hyper_project(partials)


# speedup vs baseline: 1.7640x; 1.5923x over previous
import functools

import jax
import jax.numpy as jnp
from jax import lax
from jax.experimental import pallas as pl
from jax.experimental.pallas import tpu as pltpu
from jax.experimental.pallas import tpu_sc as plsc

N_NODES = 10000
N_EDGES = 320000
D_FEAT = 128
C = 1.0
MIN_NORM = 1e-15
EPS = 1e-5

NUM_CORES = 2
NUM_SUBCORES = 16
NUM_TILES = NUM_CORES * NUM_SUBCORES
CHUNK = 112
N_CHUNKS = 90
EDGES_PER_TILE = CHUNK * N_CHUNKS
N_EDGES_PAD = EDGES_PER_TILE * NUM_TILES
ROWS_PER_TILE = 624
ROWS_TAIL = N_NODES - ROWS_PER_TILE * NUM_SUBCORES
LANES = 16
VPR = D_FEAT // LANES


def _sc_aggregate(x_tangent, row_idx, col_idx, vals, zeros_blk):
    mesh = plsc.VectorSubcoreMesh(
        core_axis_name="c", subcore_axis_name="s")

    @functools.partial(
        pl.kernel,
        out_type=jax.ShapeDtypeStruct((NUM_CORES, N_NODES, D_FEAT),
                                      jnp.float32),
        mesh=mesh,
        scratch_types=[
            pltpu.VMEM((EDGES_PER_TILE,), jnp.int32),
            pltpu.VMEM((2, CHUNK), jnp.int32),
            pltpu.VMEM((EDGES_PER_TILE,), jnp.float32),
            pltpu.VMEM((CHUNK, D_FEAT), jnp.float32),
            pltpu.VMEM((CHUNK, D_FEAT), jnp.float32),
            pltpu.VMEM_SHARED((N_NODES, D_FEAT), jnp.float32),
            pltpu.SemaphoreType.DMA,
            pltpu.SemaphoreType.DMA,
            pltpu.SemaphoreType.DMA,
            pltpu.SemaphoreType.DMA,
            pltpu.SemaphoreType.DMA,
            pltpu.SemaphoreType.DMA,
            pltpu.SemaphoreType.DMA,
        ],
    )
    def agg(x_hbm, row_hbm, col_hbm, val_hbm, zero_hbm, out_hbm,
            colv, rowv, valv, buf_a, buf_b, acc, sem_a, sem_b,
            rsem_a, rsem_b, ssem_a, ssem_b, isem):
        cid = lax.axis_index("c")
        sid = lax.axis_index("s")
        tid = sid * NUM_CORES + cid

        ebase = tid * EDGES_PER_TILE
        d1 = pltpu.async_copy(col_hbm.at[pl.ds(ebase, EDGES_PER_TILE)],
                              colv, isem)
        d3 = pltpu.async_copy(val_hbm.at[pl.ds(ebase, EDGES_PER_TILE)],
                              valv, isem)

        for _h in range(2):
            pltpu.sync_copy(
                zero_hbm.at[pl.ds(_h * (ROWS_PER_TILE // 2),
                                  ROWS_PER_TILE // 2)],
                acc.at[pl.ds(sid * ROWS_PER_TILE + _h * (ROWS_PER_TILE // 2),
                             ROWS_PER_TILE // 2)])

        @pl.when(sid == NUM_SUBCORES - 1)
        def _zero_tail():
            pltpu.sync_copy(
                zero_hbm.at[pl.ds(0, ROWS_TAIL)],
                acc.at[pl.ds(NUM_SUBCORES * ROWS_PER_TILE, ROWS_TAIL)])

        d1.wait()
        d3.wait()
        plsc.subcore_barrier()

        def row_start(i, par, rsem):
            pltpu.async_copy(row_hbm.at[pl.ds(ebase + i * CHUNK, CHUNK)],
                             rowv.at[par], rsem)

        def row_wait(i, par, rsem):
            pltpu.make_async_copy(
                row_hbm.at[pl.ds(ebase + i * CHUNK, CHUNK)],
                rowv.at[par], rsem).wait()

        def gather_start(i, buf, sem):
            pltpu.async_copy(x_hbm.at[colv.at[pl.ds(i * CHUNK, CHUNK)]],
                             buf, sem)

        def gather_wait(i, buf, sem):
            pltpu.make_async_copy(x_hbm.at[colv.at[pl.ds(i * CHUNK, CHUNK)]],
                                  buf, sem).wait()

        def scale(i, buf):
            def g(gi, c2):
                vv = valv[pl.ds(i * CHUNK + gi * LANES, LANES)]
                for j in range(LANES):
                    v = vv[j]
                    e = gi * LANES + j
                    for k in range(VPR):
                        sl = pl.ds(k * LANES, LANES)
                        buf[e, sl] = buf[e, sl] * v
                return c2

            lax.fori_loop(0, CHUNK // LANES, g, 0, unroll=False)

        def scatter_start(par, buf, ssem):
            pltpu.async_copy(buf, acc.at[rowv.at[par]], ssem, add=True)

        def scatter_wait(par, buf, ssem):
            pltpu.make_async_copy(buf, acc.at[rowv.at[par]], ssem).wait()

        gather_start(0, buf_a, sem_a)
        row_start(0, 0, rsem_a)

        def pair_body(p, carry):
            i0 = 2 * p
            i1 = 2 * p + 1

            @pl.when(p > 0)
            def _drain_b():
                scatter_wait(1, buf_b, ssem_b)

            gather_start(i1, buf_b, sem_b)
            row_start(i1, 1, rsem_b)
            gather_wait(i0, buf_a, sem_a)
            scale(i0, buf_a)
            row_wait(i0, 0, rsem_a)
            scatter_start(0, buf_a, ssem_a)
            gather_wait(i1, buf_b, sem_b)
            scale(i1, buf_b)
            scatter_wait(0, buf_a, ssem_a)

            @pl.when(i1 + 1 < N_CHUNKS)
            def _next_a():
                gather_start(i1 + 1, buf_a, sem_a)
                row_start(i1 + 1, 0, rsem_a)

            row_wait(i1, 1, rsem_b)
            scatter_start(1, buf_b, ssem_b)
            return carry

        lax.fori_loop(0, N_CHUNKS // 2, pair_body, 0, unroll=False)
        scatter_wait(1, buf_b, ssem_b)
        plsc.subcore_barrier()

        for _h in range(2):
            _sl = pl.ds(sid * ROWS_PER_TILE + _h * (ROWS_PER_TILE // 2),
                        ROWS_PER_TILE // 2)
            pltpu.sync_copy(acc.at[_sl], out_hbm.at[cid, _sl])

        @pl.when(sid == NUM_SUBCORES - 1)
        def _dump_tail():
            tl = pl.ds(NUM_SUBCORES * ROWS_PER_TILE, ROWS_TAIL)
            pltpu.sync_copy(acc.at[tl], out_hbm.at[cid, tl])

    return agg(x_tangent, row_idx, col_idx, vals, zeros_blk)


def _hyper_body(p_ref, o_ref):
    s = p_ref[0] + p_ref[1]
    sqrt_c = jnp.sqrt(C)
    nsq = jnp.sum(s * s, axis=-1, keepdims=True)
    u_norm = jnp.maximum(jnp.sqrt(nsq), MIN_NORM)
    gamma = jnp.tanh(sqrt_c * u_norm) * s / (sqrt_c * u_norm)
    gsq = jnp.sum(gamma * gamma, axis=-1, keepdims=True)
    g_norm = jnp.maximum(jnp.sqrt(gsq), MIN_NORM)
    maxnorm = (1.0 - EPS) / sqrt_c
    o_ref[...] = jnp.where(g_norm > maxnorm, gamma / g_norm * maxnorm, gamma)


def _hyper_project(partials):
    blk = 1000
    grid = N_NODES // blk
    return pl.pallas_call(
        _hyper_body,
        grid=(grid,),
        in_specs=[pl.BlockSpec((NUM_CORES, blk, D_FEAT),
                               lambda i: (0, i, 0))],
        out_specs=pl.BlockSpec((blk, D_FEAT), lambda i: (i, 0)),
        out_shape=jax.ShapeDtypeStruct((N_NODES, D_FEAT), jnp.float32),
    )(partials)


def kernel(x_tangent, adj_indices, adj_values):
    idx = adj_indices.astype(jnp.int32)
    pad = N_EDGES_PAD - N_EDGES
    spread = (jnp.arange(pad, dtype=jnp.int32) * 37) % N_NODES
    row_idx = jnp.concatenate([idx[0], spread])
    col_idx = jnp.concatenate([idx[1], spread])
    vals = jnp.concatenate([adj_values, jnp.zeros((pad,), jnp.float32)])
    zeros_blk = jnp.zeros((ROWS_PER_TILE, D_FEAT), jnp.float32)
    partials = _sc_aggregate(x_tangent, row_idx, col_idx, vals,
                             zeros_blk)
    return _hyper_project(partials)


# speedup vs baseline: 2.0324x; 1.1521x over previous
import functools

import jax
import jax.numpy as jnp
from jax import lax
from jax.experimental import pallas as pl
from jax.experimental.pallas import tpu as pltpu
from jax.experimental.pallas import tpu_sc as plsc

N_NODES = 10000
N_EDGES = 320000
D_FEAT = 128
C = 1.0
MIN_NORM = 1e-15
EPS = 1e-5

NUM_CORES = 2
NUM_SUBCORES = 16
NUM_TILES = NUM_CORES * NUM_SUBCORES
CHUNK = 48
N_CHUNKS = 212
EDGES_PER_TILE = CHUNK * N_CHUNKS
N_EDGES_PAD = EDGES_PER_TILE * NUM_TILES
ROWS_PER_TILE = 624
ROWS_TAIL = N_NODES - ROWS_PER_TILE * NUM_SUBCORES
LANES = 16
VPR = D_FEAT // LANES


def _sc_aggregate(x_tangent, row_idx, col_idx, vals, zeros_blk):
    mesh = plsc.VectorSubcoreMesh(
        core_axis_name="c", subcore_axis_name="s")

    @functools.partial(
        pl.kernel,
        out_type=jax.ShapeDtypeStruct((NUM_CORES, N_NODES, D_FEAT),
                                      jnp.float32),
        mesh=mesh,
        scratch_types=[
            pltpu.VMEM((EDGES_PER_TILE,), jnp.int32),
            pltpu.VMEM((4, CHUNK), jnp.int32),
            pltpu.VMEM((EDGES_PER_TILE,), jnp.float32),
            [pltpu.VMEM((CHUNK, D_FEAT), jnp.float32) for _ in range(4)],
            pltpu.VMEM_SHARED((N_NODES, D_FEAT), jnp.float32),
            [pltpu.SemaphoreType.DMA for _ in range(4)],
            [pltpu.SemaphoreType.DMA for _ in range(4)],
            [pltpu.SemaphoreType.DMA for _ in range(4)],
            pltpu.SemaphoreType.DMA,
        ],
    )
    def agg(x_hbm, row_hbm, col_hbm, val_hbm, zero_hbm, out_hbm,
            colv, rowv, valv, bufs, acc, gsems, rsems, ssems, isem):
        cid = lax.axis_index("c")
        sid = lax.axis_index("s")
        tid = sid * NUM_CORES + cid

        ebase = tid * EDGES_PER_TILE
        d1 = pltpu.async_copy(col_hbm.at[pl.ds(ebase, EDGES_PER_TILE)],
                              colv, isem)
        d3 = pltpu.async_copy(val_hbm.at[pl.ds(ebase, EDGES_PER_TILE)],
                              valv, isem)

        for _h in range(2):
            pltpu.sync_copy(
                zero_hbm.at[pl.ds(_h * (ROWS_PER_TILE // 2),
                                  ROWS_PER_TILE // 2)],
                acc.at[pl.ds(sid * ROWS_PER_TILE + _h * (ROWS_PER_TILE // 2),
                             ROWS_PER_TILE // 2)])

        @pl.when(sid == NUM_SUBCORES - 1)
        def _zero_tail():
            pltpu.sync_copy(
                zero_hbm.at[pl.ds(0, ROWS_TAIL)],
                acc.at[pl.ds(NUM_SUBCORES * ROWS_PER_TILE, ROWS_TAIL)])

        d1.wait()
        d3.wait()
        plsc.subcore_barrier()

        def row_start(i, par, rsem):
            pltpu.async_copy(row_hbm.at[pl.ds(ebase + i * CHUNK, CHUNK)],
                             rowv.at[par], rsem)

        def row_wait(i, par, rsem):
            pltpu.make_async_copy(
                row_hbm.at[pl.ds(ebase + i * CHUNK, CHUNK)],
                rowv.at[par], rsem).wait()

        def gather_start(i, buf, sem):
            pltpu.async_copy(x_hbm.at[colv.at[pl.ds(i * CHUNK, CHUNK)]],
                             buf, sem)

        def gather_wait(i, buf, sem):
            pltpu.make_async_copy(x_hbm.at[colv.at[pl.ds(i * CHUNK, CHUNK)]],
                                  buf, sem).wait()

        def scale(i, buf):
            def g(gi, c2):
                vv = valv[pl.ds(i * CHUNK + gi * LANES, LANES)]
                for j in range(LANES):
                    v = vv[j]
                    e = gi * LANES + j
                    for k in range(VPR):
                        sl = pl.ds(k * LANES, LANES)
                        buf[e, sl] = buf[e, sl] * v
                return c2

            lax.fori_loop(0, CHUNK // LANES, g, 0, unroll=False)

        def scatter_start(par, buf, ssem):
            pltpu.async_copy(buf, acc.at[rowv.at[par]], ssem, add=True)

        def scatter_wait(par, buf, ssem):
            pltpu.make_async_copy(buf, acc.at[rowv.at[par]], ssem).wait()

        for m in range(4):
            gather_start(m, bufs[m], gsems[m])
            row_start(m, m, rsems[m])

        def quad_body(q, carry):
            for m in range(4):
                c = 4 * q + m
                mp = (m + 3) % 4
                gather_wait(c, bufs[m], gsems[m])
                scale(c, bufs[m])

                @pl.when(c > 0)
                def _drain_prev():
                    scatter_wait(mp, bufs[mp], ssems[mp])

                    @pl.when(c + 3 < N_CHUNKS)
                    def _refill():
                        gather_start(c + 3, bufs[mp], gsems[mp])
                        row_start(c + 3, mp, rsems[mp])

                row_wait(c, m, rsems[m])
                scatter_start(m, bufs[m], ssems[m])
            return carry

        lax.fori_loop(0, N_CHUNKS // 4, quad_body, 0, unroll=False)
        scatter_wait(3, bufs[3], ssems[3])
        plsc.subcore_barrier()

        for _h in range(2):
            _sl = pl.ds(sid * ROWS_PER_TILE + _h * (ROWS_PER_TILE // 2),
                        ROWS_PER_TILE // 2)
            pltpu.sync_copy(acc.at[_sl], out_hbm.at[cid, _sl])

        @pl.when(sid == NUM_SUBCORES - 1)
        def _dump_tail():
            tl = pl.ds(NUM_SUBCORES * ROWS_PER_TILE, ROWS_TAIL)
            pltpu.sync_copy(acc.at[tl], out_hbm.at[cid, tl])

    return agg(x_tangent, row_idx, col_idx, vals, zeros_blk)


def _hyper_body(p_ref, o_ref):
    s = p_ref[0] + p_ref[1]
    sqrt_c = jnp.sqrt(C)
    nsq = jnp.sum(s * s, axis=-1, keepdims=True)
    u_norm = jnp.maximum(jnp.sqrt(nsq), MIN_NORM)
    gamma = jnp.tanh(sqrt_c * u_norm) * s / (sqrt_c * u_norm)
    gsq = jnp.sum(gamma * gamma, axis=-1, keepdims=True)
    g_norm = jnp.maximum(jnp.sqrt(gsq), MIN_NORM)
    maxnorm = (1.0 - EPS) / sqrt_c
    o_ref[...] = jnp.where(g_norm > maxnorm, gamma / g_norm * maxnorm, gamma)


def _hyper_project(partials):
    blk = 1000
    grid = N_NODES // blk
    return pl.pallas_call(
        _hyper_body,
        grid=(grid,),
        in_specs=[pl.BlockSpec((NUM_CORES, blk, D_FEAT),
                               lambda i: (0, i, 0))],
        out_specs=pl.BlockSpec((blk, D_FEAT), lambda i: (i, 0)),
        out_shape=jax.ShapeDtypeStruct((N_NODES, D_FEAT), jnp.float32),
    )(partials)


def kernel(x_tangent, adj_indices, adj_values):
    idx = adj_indices.astype(jnp.int32)
    pad = N_EDGES_PAD - N_EDGES
    spread = (jnp.arange(pad, dtype=jnp.int32) * 37) % N_NODES
    row_idx = jnp.concatenate([idx[0], spread])
    col_idx = jnp.concatenate([idx[1], spread])
    vals = jnp.concatenate([adj_values, jnp.zeros((pad,), jnp.float32)])
    zeros_blk = jnp.zeros((ROWS_PER_TILE, D_FEAT), jnp.float32)
    partials = _sc_aggregate(x_tangent, row_idx, col_idx, vals,
                             zeros_blk)
    return _hyper_project(partials)


# speedup vs baseline: 2.0580x; 1.0126x over previous
import functools

import jax
import jax.numpy as jnp
from jax import lax
from jax.experimental import pallas as pl
from jax.experimental.pallas import tpu as pltpu
from jax.experimental.pallas import tpu_sc as plsc

N_NODES = 10000
N_EDGES = 320000
D_FEAT = 128
C = 1.0
MIN_NORM = 1e-15
EPS = 1e-5

NUM_CORES = 2
NUM_SUBCORES = 16
NUM_TILES = NUM_CORES * NUM_SUBCORES
CHUNK = 32
N_CHUNKS = 318
EDGES_PER_TILE = CHUNK * N_CHUNKS
N_EDGES_PAD = EDGES_PER_TILE * NUM_TILES
ROWS_PER_TILE = 624
ROWS_TAIL = N_NODES - ROWS_PER_TILE * NUM_SUBCORES
LANES = 16
VPR = D_FEAT // LANES


def _sc_aggregate(x_tangent, row_idx, col_idx, vals, zeros_blk):
    mesh = plsc.VectorSubcoreMesh(
        core_axis_name="c", subcore_axis_name="s")

    @functools.partial(
        pl.kernel,
        out_type=jax.ShapeDtypeStruct((NUM_CORES, N_NODES, D_FEAT),
                                      jnp.float32),
        mesh=mesh,
        scratch_types=[
            pltpu.VMEM((EDGES_PER_TILE,), jnp.int32),
            pltpu.VMEM((6, CHUNK), jnp.int32),
            pltpu.VMEM((EDGES_PER_TILE,), jnp.float32),
            [pltpu.VMEM((CHUNK, D_FEAT), jnp.float32) for _ in range(6)],
            pltpu.VMEM_SHARED((N_NODES, D_FEAT), jnp.float32),
            [pltpu.SemaphoreType.DMA for _ in range(6)],
            [pltpu.SemaphoreType.DMA for _ in range(6)],
            [pltpu.SemaphoreType.DMA for _ in range(6)],
            pltpu.SemaphoreType.DMA,
        ],
    )
    def agg(x_hbm, row_hbm, col_hbm, val_hbm, zero_hbm, out_hbm,
            colv, rowv, valv, bufs, acc, gsems, rsems, ssems, isem):
        cid = lax.axis_index("c")
        sid = lax.axis_index("s")
        tid = sid * NUM_CORES + cid

        ebase = tid * EDGES_PER_TILE
        d1 = pltpu.async_copy(col_hbm.at[pl.ds(ebase, EDGES_PER_TILE)],
                              colv, isem)
        d3 = pltpu.async_copy(val_hbm.at[pl.ds(ebase, EDGES_PER_TILE)],
                              valv, isem)

        for _h in range(2):
            pltpu.sync_copy(
                zero_hbm.at[pl.ds(_h * (ROWS_PER_TILE // 2),
                                  ROWS_PER_TILE // 2)],
                acc.at[pl.ds(sid * ROWS_PER_TILE + _h * (ROWS_PER_TILE // 2),
                             ROWS_PER_TILE // 2)])

        @pl.when(sid == NUM_SUBCORES - 1)
        def _zero_tail():
            pltpu.sync_copy(
                zero_hbm.at[pl.ds(0, ROWS_TAIL)],
                acc.at[pl.ds(NUM_SUBCORES * ROWS_PER_TILE, ROWS_TAIL)])

        d1.wait()
        d3.wait()
        plsc.subcore_barrier()

        def row_start(i, par, rsem):
            pltpu.async_copy(row_hbm.at[pl.ds(ebase + i * CHUNK, CHUNK)],
                             rowv.at[par], rsem)

        def row_wait(i, par, rsem):
            pltpu.make_async_copy(
                row_hbm.at[pl.ds(ebase + i * CHUNK, CHUNK)],
                rowv.at[par], rsem).wait()

        def gather_start(i, buf, sem):
            pltpu.async_copy(x_hbm.at[colv.at[pl.ds(i * CHUNK, CHUNK)]],
                             buf, sem)

        def gather_wait(i, buf, sem):
            pltpu.make_async_copy(x_hbm.at[colv.at[pl.ds(i * CHUNK, CHUNK)]],
                                  buf, sem).wait()

        def scale(i, buf):
            def g(gi, c2):
                vv = valv[pl.ds(i * CHUNK + gi * LANES, LANES)]
                for j in range(LANES):
                    v = vv[j]
                    e = gi * LANES + j
                    for k in range(VPR):
                        sl = pl.ds(k * LANES, LANES)
                        buf[e, sl] = buf[e, sl] * v
                return c2

            lax.fori_loop(0, CHUNK // LANES, g, 0, unroll=False)

        def scatter_start(par, buf, ssem):
            pltpu.async_copy(buf, acc.at[rowv.at[par]], ssem, add=True)

        def scatter_wait(par, buf, ssem):
            pltpu.make_async_copy(buf, acc.at[rowv.at[par]], ssem).wait()

        for m in range(6):
            gather_start(m, bufs[m], gsems[m])
            row_start(m, m, rsems[m])

        def quad_body(q, carry):
            for m in range(6):
                c = 6 * q + m
                mp = (m + 5) % 6
                gather_wait(c, bufs[m], gsems[m])
                scale(c, bufs[m])

                @pl.when(c > 0)
                def _drain_prev():
                    scatter_wait(mp, bufs[mp], ssems[mp])

                    @pl.when(c + 5 < N_CHUNKS)
                    def _refill():
                        gather_start(c + 5, bufs[mp], gsems[mp])
                        row_start(c + 5, mp, rsems[mp])

                row_wait(c, m, rsems[m])
                scatter_start(m, bufs[m], ssems[m])
            return carry

        lax.fori_loop(0, N_CHUNKS // 6, quad_body, 0, unroll=False)
        scatter_wait(5, bufs[5], ssems[5])
        plsc.subcore_barrier()

        for _h in range(2):
            _sl = pl.ds(sid * ROWS_PER_TILE + _h * (ROWS_PER_TILE // 2),
                        ROWS_PER_TILE // 2)
            pltpu.sync_copy(acc.at[_sl], out_hbm.at[cid, _sl])

        @pl.when(sid == NUM_SUBCORES - 1)
        def _dump_tail():
            tl = pl.ds(NUM_SUBCORES * ROWS_PER_TILE, ROWS_TAIL)
            pltpu.sync_copy(acc.at[tl], out_hbm.at[cid, tl])

    return agg(x_tangent, row_idx, col_idx, vals, zeros_blk)


def _hyper_body(p_ref, o_ref):
    s = p_ref[0] + p_ref[1]
    sqrt_c = jnp.sqrt(C)
    nsq = jnp.sum(s * s, axis=-1, keepdims=True)
    u_norm = jnp.maximum(jnp.sqrt(nsq), MIN_NORM)
    gamma = jnp.tanh(sqrt_c * u_norm) * s / (sqrt_c * u_norm)
    gsq = jnp.sum(gamma * gamma, axis=-1, keepdims=True)
    g_norm = jnp.maximum(jnp.sqrt(gsq), MIN_NORM)
    maxnorm = (1.0 - EPS) / sqrt_c
    o_ref[...] = jnp.where(g_norm > maxnorm, gamma / g_norm * maxnorm, gamma)


def _hyper_project(partials):
    blk = 1000
    grid = N_NODES // blk
    return pl.pallas_call(
        _hyper_body,
        grid=(grid,),
        in_specs=[pl.BlockSpec((NUM_CORES, blk, D_FEAT),
                               lambda i: (0, i, 0))],
        out_specs=pl.BlockSpec((blk, D_FEAT), lambda i: (i, 0)),
        out_shape=jax.ShapeDtypeStruct((N_NODES, D_FEAT), jnp.float32),
    )(partials)


def kernel(x_tangent, adj_indices, adj_values):
    idx = adj_indices.astype(jnp.int32)
    pad = N_EDGES_PAD - N_EDGES
    spread = (jnp.arange(pad, dtype=jnp.int32) * 37) % N_NODES
    row_idx = jnp.concatenate([idx[0], spread])
    col_idx = jnp.concatenate([idx[1], spread])
    vals = jnp.concatenate([adj_values, jnp.zeros((pad,), jnp.float32)])
    zeros_blk = jnp.zeros((ROWS_PER_TILE, D_FEAT), jnp.float32)
    partials = _sc_aggregate(x_tangent, row_idx, col_idx, vals,
                             zeros_blk)
    return _hyper_project(partials)


# speedup vs baseline: 2.0585x; 1.0002x over previous
import functools

import jax
import jax.numpy as jnp
from jax import lax
from jax.experimental import pallas as pl
from jax.experimental.pallas import tpu as pltpu
from jax.experimental.pallas import tpu_sc as plsc

N_NODES = 10000
N_EDGES = 320000
D_FEAT = 128
C = 1.0
MIN_NORM = 1e-15
EPS = 1e-5

NUM_CORES = 2
NUM_SUBCORES = 16
NUM_TILES = NUM_CORES * NUM_SUBCORES
CHUNK = 32
N_CHUNKS = 315
EDGES_PER_TILE = CHUNK * N_CHUNKS
N_EDGES_PAD = EDGES_PER_TILE * NUM_TILES
ROWS_PER_TILE = 624
ROWS_TAIL = N_NODES - ROWS_PER_TILE * NUM_SUBCORES
LANES = 16
VPR = D_FEAT // LANES


def _sc_aggregate(x_tangent, row_idx, col_idx, vals, zeros_blk):
    mesh = plsc.VectorSubcoreMesh(
        core_axis_name="c", subcore_axis_name="s")

    @functools.partial(
        pl.kernel,
        out_type=jax.ShapeDtypeStruct((NUM_CORES, N_NODES, D_FEAT),
                                      jnp.float32),
        mesh=mesh,
        scratch_types=[
            pltpu.VMEM((EDGES_PER_TILE,), jnp.int32),
            pltpu.VMEM((7, CHUNK), jnp.int32),
            pltpu.VMEM((EDGES_PER_TILE,), jnp.float32),
            [pltpu.VMEM((CHUNK, D_FEAT), jnp.float32) for _ in range(7)],
            pltpu.VMEM_SHARED((N_NODES, D_FEAT), jnp.float32),
            [pltpu.SemaphoreType.DMA for _ in range(7)],
            [pltpu.SemaphoreType.DMA for _ in range(7)],
            [pltpu.SemaphoreType.DMA for _ in range(7)],
            pltpu.SemaphoreType.DMA,
        ],
    )
    def agg(x_hbm, row_hbm, col_hbm, val_hbm, zero_hbm, out_hbm,
            colv, rowv, valv, bufs, acc, gsems, rsems, ssems, isem):
        cid = lax.axis_index("c")
        sid = lax.axis_index("s")
        tid = sid * NUM_CORES + cid

        ebase = tid * EDGES_PER_TILE
        d1 = pltpu.async_copy(col_hbm.at[pl.ds(ebase, EDGES_PER_TILE)],
                              colv, isem)
        d3 = pltpu.async_copy(val_hbm.at[pl.ds(ebase, EDGES_PER_TILE)],
                              valv, isem)

        for _h in range(2):
            pltpu.sync_copy(
                zero_hbm.at[pl.ds(_h * (ROWS_PER_TILE // 2),
                                  ROWS_PER_TILE // 2)],
                acc.at[pl.ds(sid * ROWS_PER_TILE + _h * (ROWS_PER_TILE // 2),
                             ROWS_PER_TILE // 2)])

        @pl.when(sid == NUM_SUBCORES - 1)
        def _zero_tail():
            pltpu.sync_copy(
                zero_hbm.at[pl.ds(0, ROWS_TAIL)],
                acc.at[pl.ds(NUM_SUBCORES * ROWS_PER_TILE, ROWS_TAIL)])

        d1.wait()
        d3.wait()
        plsc.subcore_barrier()

        def row_start(i, par, rsem):
            pltpu.async_copy(row_hbm.at[pl.ds(ebase + i * CHUNK, CHUNK)],
                             rowv.at[par], rsem)

        def row_wait(i, par, rsem):
            pltpu.make_async_copy(
                row_hbm.at[pl.ds(ebase + i * CHUNK, CHUNK)],
                rowv.at[par], rsem).wait()

        def gather_start(i, buf, sem):
            pltpu.async_copy(x_hbm.at[colv.at[pl.ds(i * CHUNK, CHUNK)]],
                             buf, sem)

        def gather_wait(i, buf, sem):
            pltpu.make_async_copy(x_hbm.at[colv.at[pl.ds(i * CHUNK, CHUNK)]],
                                  buf, sem).wait()

        def scale(i, buf):
            def g(gi, c2):
                vv = valv[pl.ds(i * CHUNK + gi * LANES, LANES)]
                for j in range(LANES):
                    v = vv[j]
                    e = gi * LANES + j
                    for k in range(VPR):
                        sl = pl.ds(k * LANES, LANES)
                        buf[e, sl] = buf[e, sl] * v
                return c2

            lax.fori_loop(0, CHUNK // LANES, g, 0, unroll=False)

        def scatter_start(par, buf, ssem):
            pltpu.async_copy(buf, acc.at[rowv.at[par]], ssem, add=True)

        def scatter_wait(par, buf, ssem):
            pltpu.make_async_copy(buf, acc.at[rowv.at[par]], ssem).wait()

        for m in range(7):
            gather_start(m, bufs[m], gsems[m])
            row_start(m, m, rsems[m])

        def quad_body(q, carry):
            for m in range(7):
                c = 7 * q + m
                mp = (m + 6) % 7
                gather_wait(c, bufs[m], gsems[m])
                scale(c, bufs[m])

                @pl.when(c > 0)
                def _drain_prev():
                    scatter_wait(mp, bufs[mp], ssems[mp])

                    @pl.when(c + 6 < N_CHUNKS)
                    def _refill():
                        gather_start(c + 6, bufs[mp], gsems[mp])
                        row_start(c + 6, mp, rsems[mp])

                row_wait(c, m, rsems[m])
                scatter_start(m, bufs[m], ssems[m])
            return carry

        lax.fori_loop(0, N_CHUNKS // 7, quad_body, 0, unroll=False)
        scatter_wait(6, bufs[6], ssems[6])
        plsc.subcore_barrier()

        for _h in range(2):
            _sl = pl.ds(sid * ROWS_PER_TILE + _h * (ROWS_PER_TILE // 2),
                        ROWS_PER_TILE // 2)
            pltpu.sync_copy(acc.at[_sl], out_hbm.at[cid, _sl])

        @pl.when(sid == NUM_SUBCORES - 1)
        def _dump_tail():
            tl = pl.ds(NUM_SUBCORES * ROWS_PER_TILE, ROWS_TAIL)
            pltpu.sync_copy(acc.at[tl], out_hbm.at[cid, tl])

    return agg(x_tangent, row_idx, col_idx, vals, zeros_blk)


def _hyper_body(p_ref, o_ref):
    s = p_ref[0] + p_ref[1]
    sqrt_c = jnp.sqrt(C)
    nsq = jnp.sum(s * s, axis=-1, keepdims=True)
    u_norm = jnp.maximum(jnp.sqrt(nsq), MIN_NORM)
    gamma = jnp.tanh(sqrt_c * u_norm) * s / (sqrt_c * u_norm)
    gsq = jnp.sum(gamma * gamma, axis=-1, keepdims=True)
    g_norm = jnp.maximum(jnp.sqrt(gsq), MIN_NORM)
    maxnorm = (1.0 - EPS) / sqrt_c
    o_ref[...] = jnp.where(g_norm > maxnorm, gamma / g_norm * maxnorm, gamma)


def _hyper_project(partials):
    blk = 1000
    grid = N_NODES // blk
    return pl.pallas_call(
        _hyper_body,
        grid=(grid,),
        in_specs=[pl.BlockSpec((NUM_CORES, blk, D_FEAT),
                               lambda i: (0, i, 0))],
        out_specs=pl.BlockSpec((blk, D_FEAT), lambda i: (i, 0)),
        out_shape=jax.ShapeDtypeStruct((N_NODES, D_FEAT), jnp.float32),
    )(partials)


def kernel(x_tangent, adj_indices, adj_values):
    idx = adj_indices.astype(jnp.int32)
    pad = N_EDGES_PAD - N_EDGES
    spread = (jnp.arange(pad, dtype=jnp.int32) * 37) % N_NODES
    row_idx = jnp.concatenate([idx[0], spread])
    col_idx = jnp.concatenate([idx[1], spread])
    vals = jnp.concatenate([adj_values, jnp.zeros((pad,), jnp.float32)])
    zeros_blk = jnp.zeros((ROWS_PER_TILE, D_FEAT), jnp.float32)
    partials = _sc_aggregate(x_tangent, row_idx, col_idx, vals,
                             zeros_blk)
    return _hyper_project(partials)


# speedup vs baseline: 2.0612x; 1.0013x over previous
import functools

import jax
import jax.numpy as jnp
from jax import lax
from jax.experimental import pallas as pl
from jax.experimental.pallas import tpu as pltpu
from jax.experimental.pallas import tpu_sc as plsc

N_NODES = 10000
N_EDGES = 320000
D_FEAT = 128
C = 1.0
MIN_NORM = 1e-15
EPS = 1e-5

NUM_CORES = 2
NUM_SUBCORES = 16
NUM_TILES = NUM_CORES * NUM_SUBCORES
CHUNK = 32
N_CHUNKS = 315
EDGES_PER_TILE = CHUNK * N_CHUNKS
N_EDGES_PAD = EDGES_PER_TILE * NUM_TILES
ROWS_PER_TILE = 624
ROWS_TAIL = N_NODES - ROWS_PER_TILE * NUM_SUBCORES
LANES = 16
VPR = D_FEAT // LANES


def _sc_aggregate(x_tangent, row_idx, col_idx, vals, zeros_blk):
    mesh = plsc.VectorSubcoreMesh(
        core_axis_name="c", subcore_axis_name="s")

    @functools.partial(
        pl.kernel,
        out_type=jax.ShapeDtypeStruct((NUM_CORES, N_NODES, D_FEAT),
                                      jnp.float32),
        mesh=mesh,
        scratch_types=[
            pltpu.VMEM((EDGES_PER_TILE,), jnp.int32),
            pltpu.VMEM((7, CHUNK), jnp.int32),
            pltpu.VMEM((EDGES_PER_TILE,), jnp.float32),
            [pltpu.VMEM((CHUNK, D_FEAT), jnp.float32) for _ in range(7)],
            pltpu.VMEM_SHARED((N_NODES, D_FEAT), jnp.float32),
            [pltpu.SemaphoreType.DMA for _ in range(7)],
            [pltpu.SemaphoreType.DMA for _ in range(7)],
            [pltpu.SemaphoreType.DMA for _ in range(7)],
            pltpu.SemaphoreType.DMA,
        ],
    )
    def agg(x_hbm, row_hbm, col_hbm, val_hbm, zero_hbm, out_hbm,
            colv, rowv, valv, bufs, acc, gsems, rsems, ssems, isem):
        cid = lax.axis_index("c")
        sid = lax.axis_index("s")
        tid = sid * NUM_CORES + cid

        ebase = tid * EDGES_PER_TILE
        d1 = pltpu.async_copy(col_hbm.at[pl.ds(ebase, EDGES_PER_TILE)],
                              colv, isem)
        d3 = pltpu.async_copy(val_hbm.at[pl.ds(ebase, EDGES_PER_TILE)],
                              valv, isem)

        for _h in range(2):
            pltpu.sync_copy(
                zero_hbm.at[pl.ds(_h * (ROWS_PER_TILE // 2),
                                  ROWS_PER_TILE // 2)],
                acc.at[pl.ds(sid * ROWS_PER_TILE + _h * (ROWS_PER_TILE // 2),
                             ROWS_PER_TILE // 2)])

        @pl.when(sid == NUM_SUBCORES - 1)
        def _zero_tail():
            pltpu.sync_copy(
                zero_hbm.at[pl.ds(0, ROWS_TAIL)],
                acc.at[pl.ds(NUM_SUBCORES * ROWS_PER_TILE, ROWS_TAIL)])

        d1.wait()
        d3.wait()
        plsc.subcore_barrier()

        def row_start(i, par, rsem):
            pltpu.async_copy(row_hbm.at[pl.ds(ebase + i * CHUNK, CHUNK)],
                             rowv.at[par], rsem)

        def row_wait(i, par, rsem):
            pltpu.make_async_copy(
                row_hbm.at[pl.ds(ebase + i * CHUNK, CHUNK)],
                rowv.at[par], rsem).wait()

        def gather_start(i, buf, sem):
            pltpu.async_copy(x_hbm.at[colv.at[pl.ds(i * CHUNK, CHUNK)]],
                             buf, sem)

        def gather_wait(i, buf, sem):
            pltpu.make_async_copy(x_hbm.at[colv.at[pl.ds(i * CHUNK, CHUNK)]],
                                  buf, sem).wait()

        def scale(i, buf):
            def g(gi, c2):
                vv = valv[pl.ds(i * CHUNK + gi * LANES, LANES)]
                for j in range(LANES):
                    v = vv[j]
                    e = gi * LANES + j
                    for k in range(VPR):
                        sl = pl.ds(k * LANES, LANES)
                        buf[e, sl] = buf[e, sl] * v
                return c2

            lax.fori_loop(0, CHUNK // LANES, g, 0, unroll=False)

        def scatter_start(par, buf, ssem):
            pltpu.async_copy(buf, acc.at[rowv.at[par]], ssem, add=True)

        def scatter_wait(par, buf, ssem):
            pltpu.make_async_copy(buf, acc.at[rowv.at[par]], ssem).wait()

        for m in range(7):
            gather_start(m, bufs[m], gsems[m])
            row_start(m, m, rsems[m])

        def chunk_body(q, carry):
            for m in range(7):
                c = 7 * q + m
                mp = (m + 6) % 7
                gather_wait(c, bufs[m], gsems[m])
                scale(c, bufs[m])

                @pl.when(c > 0)
                def _drain_prev():
                    scatter_wait(mp, bufs[mp], ssems[mp])

                    @pl.when(c + 6 < N_CHUNKS)
                    def _refill():
                        gather_start(c + 6, bufs[mp], gsems[mp])
                        row_start(c + 6, mp, rsems[mp])

                row_wait(c, m, rsems[m])
                scatter_start(m, bufs[m], ssems[m])
            return carry

        lax.fori_loop(0, N_CHUNKS // 7, chunk_body, 0, unroll=False)
        scatter_wait(6, bufs[6], ssems[6])
        plsc.subcore_barrier()

        for _h in range(2):
            _sl = pl.ds(sid * ROWS_PER_TILE + _h * (ROWS_PER_TILE // 2),
                        ROWS_PER_TILE // 2)
            pltpu.sync_copy(acc.at[_sl], out_hbm.at[cid, _sl])

        @pl.when(sid == NUM_SUBCORES - 1)
        def _dump_tail():
            tl = pl.ds(NUM_SUBCORES * ROWS_PER_TILE, ROWS_TAIL)
            pltpu.sync_copy(acc.at[tl], out_hbm.at[cid, tl])

    return agg(x_tangent, row_idx, col_idx, vals, zeros_blk)


def _hyper_body(p_ref, o_ref):
    s = p_ref[0] + p_ref[1]
    sqrt_c = jnp.sqrt(C)
    nsq = jnp.sum(s * s, axis=-1, keepdims=True)
    u_norm = jnp.maximum(jnp.sqrt(nsq), MIN_NORM)
    gamma = jnp.tanh(sqrt_c * u_norm) * s / (sqrt_c * u_norm)
    gsq = jnp.sum(gamma * gamma, axis=-1, keepdims=True)
    g_norm = jnp.maximum(jnp.sqrt(gsq), MIN_NORM)
    maxnorm = (1.0 - EPS) / sqrt_c
    o_ref[...] = jnp.where(g_norm > maxnorm, gamma / g_norm * maxnorm, gamma)


def _hyper_project(partials):
    blk = 1000
    grid = N_NODES // blk
    return pl.pallas_call(
        _hyper_body,
        grid=(grid,),
        in_specs=[pl.BlockSpec((NUM_CORES, blk, D_FEAT),
                               lambda i: (0, i, 0))],
        out_specs=pl.BlockSpec((blk, D_FEAT), lambda i: (i, 0)),
        out_shape=jax.ShapeDtypeStruct((N_NODES, D_FEAT), jnp.float32),
    )(partials)


def kernel(x_tangent, adj_indices, adj_values):
    idx = adj_indices.astype(jnp.int32)
    pad = N_EDGES_PAD - N_EDGES
    spread = (jnp.arange(pad, dtype=jnp.int32) * 37) % N_NODES
    row_idx = jnp.concatenate([idx[0], spread])
    col_idx = jnp.concatenate([idx[1], spread])
    vals = jnp.concatenate([adj_values, jnp.zeros((pad,), jnp.float32)])
    zeros_blk = jnp.zeros((ROWS_PER_TILE, D_FEAT), jnp.float32)
    partials = _sc_aggregate(x_tangent, row_idx, col_idx, vals,
                             zeros_blk)
    return _hyper_project(partials)
